# Initial kernel scaffold; baseline (speedup 1.0000x reference)
import jax
import jax.numpy as jnp
from jax.experimental import pallas as pl


def kernel(metric, W1, b1, W2, b2):
    raise NotImplementedError("write your pallas kernel here")


# speedup vs baseline: 3.4685x; 3.4685x over previous
import functools

import jax
import jax.numpy as jnp
from jax.experimental import pallas as pl


def _stage1_body(m_ref, sim_ref, cs_ref, *, C):
    t = pl.program_id(1)
    x = m_ref[0]
    a = x[:, :C]
    b = x[:, C:]
    dot = jnp.sum(a * b, axis=1, keepdims=True)
    na = jnp.sqrt(jnp.sum(a * a, axis=1, keepdims=True))
    nb = jnp.sqrt(jnp.sum(b * b, axis=1, keepdims=True))
    denom = jnp.maximum(na, 1e-12) * jnp.maximum(nb, 1e-12)
    sim_ref[0] = dot / denom
    colpart = jnp.sum(x, axis=0, keepdims=True)
    g = colpart[:, :C] + colpart[:, C:]

    @pl.when(t == 0)
    def _init():
        cs_ref[0] = g

    @pl.when(t != 0)
    def _acc():
        cs_ref[0] += g


def _stage2_body(cs_ref, sim_ref, w1_ref, b1_ref, w2_ref, b2_ref,
                 logits_ref, mask_ref, theta_ref, ratio_ref, mpm_ref, kre_ref,
                 *, T, tau, theta_min, theta_max):
    g = cs_ref[...] * (1.0 / T)
    h = jnp.dot(g, w1_ref[...], preferred_element_type=jnp.float32) + b1_ref[...]
    h = 0.5 * h * (1.0 + jax.lax.erf(h * jnp.float32(0.7071067811865476)))
    t2 = jnp.dot(h, w2_ref[...], preferred_element_type=jnp.float32) + b2_ref[...]
    theta = theta_min + (theta_max - theta_min) * jax.nn.sigmoid(t2)
    theta_ref[...] = theta
    logits = (sim_ref[...] - theta) / max(tau, 1e-6)
    logits_ref[...] = logits
    maskf = (logits >= 0).astype(jnp.float32)
    mask_ref[...] = maskf
    n = logits.shape[0] * logits.shape[1]
    ratio = jnp.sum(maskf, axis=(0, 1), keepdims=True) * (1.0 / n)
    ratio_ref[...] = ratio
    mpm_ref[...] = jnp.sum(jax.nn.sigmoid(logits), axis=(0, 1), keepdims=True) * (1.0 / n)
    kre_ref[...] = 1.0 - 0.5 * ratio


def kernel(metric, W1, b1, W2, b2):
    tau_gate = 0.1
    theta_min = 0.0
    theta_max = 2.0
    B, T, C = metric.shape
    if T % 2 == 1:
        metric = metric[:, :-1, :]
        T = T - 1
    P = T // 2
    H = W1.shape[1]

    PT = 256
    while P % PT != 0:
        PT //= 2
    NT = P // PT

    m2 = metric.reshape(B, P, 2 * C)

    sim3, colsum = pl.pallas_call(
        functools.partial(_stage1_body, C=C),
        grid=(B, NT),
        in_specs=[pl.BlockSpec((1, PT, 2 * C), lambda b, t: (b, t, 0))],
        out_specs=[
            pl.BlockSpec((1, PT, 1), lambda b, t: (b, t, 0)),
            pl.BlockSpec((1, 1, C), lambda b, t: (b, 0, 0)),
        ],
        out_shape=[
            jax.ShapeDtypeStruct((B, P, 1), jnp.float32),
            jax.ShapeDtypeStruct((B, 1, C), jnp.float32),
        ],
    )(m2)

    sim = sim3.reshape(B, P)
    colsum = colsum.reshape(B, C)

    outs = pl.pallas_call(
        functools.partial(_stage2_body, T=T, tau=tau_gate,
                          theta_min=theta_min, theta_max=theta_max),
        out_shape=[
            jax.ShapeDtypeStruct((B, P), jnp.float32),
            jax.ShapeDtypeStruct((B, P), jnp.float32),
            jax.ShapeDtypeStruct((B, 1), jnp.float32),
            jax.ShapeDtypeStruct((1, 1), jnp.float32),
            jax.ShapeDtypeStruct((1, 1), jnp.float32),
            jax.ShapeDtypeStruct((1, 1), jnp.float32),
        ],
    )(colsum, sim, W1, b1.reshape(1, H), W2, b2.reshape(1, 1))

    logits, maskf, theta2, ratio, mpm, kre = outs
    return (logits,
            maskf.astype(bool),
            theta2.reshape(B),
            ratio.reshape(()),
            mpm.reshape(()),
            kre.reshape(()))


# speedup vs baseline: 8.0270x; 2.3142x over previous
import functools

import jax
import jax.numpy as jnp
from jax.experimental import pallas as pl
from jax.experimental.pallas import tpu as pltpu


def _stage1_body(m_ref, sim_ref, cs_ref, *, C):
    t = pl.program_id(1)
    x = m_ref[0]
    tt = x.shape[0]
    xs = pltpu.roll(x, tt - 1, 0)
    dotf = jnp.sum(x * xs, axis=1, keepdims=True)
    n2 = jnp.sum(x * x, axis=1, keepdims=True)
    nrm = jnp.maximum(jnp.sqrt(n2), 1e-12)
    denom = nrm * pltpu.roll(nrm, tt - 1, 0)
    sim_ref[0] = dotf / denom
    g = jnp.sum(x, axis=0, keepdims=True)

    @pl.when(t == 0)
    def _init():
        cs_ref[0] = g

    @pl.when(t != 0)
    def _acc():
        cs_ref[0] += g


def _stage2_body(cs_ref, sim_ref, w1_ref, b1_ref, w2_ref, b2_ref,
                 logits_ref, mask_ref, theta_ref, ratio_ref, mpm_ref, kre_ref,
                 *, T, tau, theta_min, theta_max):
    g = cs_ref[...] * (1.0 / T)
    h = jnp.dot(g, w1_ref[...], preferred_element_type=jnp.float32) + b1_ref[...]
    h = 0.5 * h * (1.0 + jax.lax.erf(h * jnp.float32(0.7071067811865476)))
    t2 = jnp.dot(h, w2_ref[...], preferred_element_type=jnp.float32) + b2_ref[...]
    theta = theta_min + (theta_max - theta_min) * jax.nn.sigmoid(t2)
    theta_ref[...] = theta
    logits = (sim_ref[...] - theta) / max(tau, 1e-6)
    logits_ref[...] = logits
    maskf = (logits >= 0).astype(jnp.float32)
    mask_ref[...] = maskf
    n = logits.shape[0] * logits.shape[1]
    ratio = jnp.sum(maskf, axis=(0, 1), keepdims=True) * (1.0 / n)
    ratio_ref[...] = ratio
    mpm_ref[...] = jnp.sum(jax.nn.sigmoid(logits), axis=(0, 1), keepdims=True) * (1.0 / n)
    kre_ref[...] = 1.0 - 0.5 * ratio


def kernel(metric, W1, b1, W2, b2):
    tau_gate = 0.1
    theta_min = 0.0
    theta_max = 2.0
    B, T, C = metric.shape
    if T % 2 == 1:
        metric = metric[:, :-1, :]
        T = T - 1
    P = T // 2
    H = W1.shape[1]

    PT = 256
    while P % PT != 0:
        PT //= 2
    NT = P // PT
    TT = 2 * PT

    sim3, colsum = pl.pallas_call(
        functools.partial(_stage1_body, C=C),
        grid=(B, NT),
        in_specs=[pl.BlockSpec((1, TT, C), lambda b, t: (b, t, 0))],
        out_specs=[
            pl.BlockSpec((1, TT, 1), lambda b, t: (b, t, 0)),
            pl.BlockSpec((1, 1, C), lambda b, t: (b, 0, 0)),
        ],
        out_shape=[
            jax.ShapeDtypeStruct((B, T, 1), jnp.float32),
            jax.ShapeDtypeStruct((B, 1, C), jnp.float32),
        ],
    )(metric)

    sim = sim3.reshape(B, T)[:, ::2]
    colsum = colsum.reshape(B, C)

    outs = pl.pallas_call(
        functools.partial(_stage2_body, T=T, tau=tau_gate,
                          theta_min=theta_min, theta_max=theta_max),
        out_shape=[
            jax.ShapeDtypeStruct((B, P), jnp.float32),
            jax.ShapeDtypeStruct((B, P), jnp.float32),
            jax.ShapeDtypeStruct((B, 1), jnp.float32),
            jax.ShapeDtypeStruct((1, 1), jnp.float32),
            jax.ShapeDtypeStruct((1, 1), jnp.float32),
            jax.ShapeDtypeStruct((1, 1), jnp.float32),
        ],
    )(colsum, sim, W1, b1.reshape(1, H), W2, b2.reshape(1, 1))

    logits, maskf, theta2, ratio, mpm, kre = outs
    return (logits,
            maskf.astype(bool),
            theta2.reshape(B),
            ratio.reshape(()),
            mpm.reshape(()),
            kre.reshape(()))


# speedup vs baseline: 9.9113x; 1.2348x over previous
import functools

import jax
import jax.numpy as jnp
from jax.experimental import pallas as pl
from jax.experimental.pallas import tpu as pltpu


def _stage1_body(m_ref, sim_ref, cs_ref, *, C):
    t = pl.program_id(1)
    x = m_ref[0]
    tt = x.shape[0]
    xs = pltpu.roll(x, tt - 1, 0)
    dotf = jnp.sum(x * xs, axis=1, keepdims=True)
    n2 = jnp.sum(x * x, axis=1, keepdims=True)
    nrm = jnp.maximum(jnp.sqrt(n2), 1e-12)
    denom = nrm * pltpu.roll(nrm, tt - 1, 0)
    sim_ref[0] = dotf / denom
    g = jnp.sum(x, axis=0, keepdims=True)

    @pl.when(t == 0)
    def _init():
        cs_ref[0] = g

    @pl.when(t != 0)
    def _acc():
        cs_ref[0] += g


def _stage2_body(cs_ref, sim_ref, w1_ref, b1_ref, w2_ref, b2_ref,
                 logits_ref, mask_ref, theta_ref, ratio_ref, mpm_ref, kre_ref,
                 *, T, tau, theta_min, theta_max):
    g = cs_ref[...] * (1.0 / T)
    h = jnp.dot(g, w1_ref[...], preferred_element_type=jnp.float32) + b1_ref[...]
    h = 0.5 * h * (1.0 + jax.lax.erf(h * jnp.float32(0.7071067811865476)))
    t2 = jnp.dot(h, w2_ref[...], preferred_element_type=jnp.float32) + b2_ref[...]
    theta = theta_min + (theta_max - theta_min) * jax.nn.sigmoid(t2)
    theta_ref[...] = theta
    logits = (sim_ref[...] - theta) / max(tau, 1e-6)
    logits_ref[...] = logits
    maskf = (logits >= 0).astype(jnp.float32)
    mask_ref[...] = maskf
    n = logits.shape[0] * logits.shape[1]
    ratio = jnp.sum(maskf, axis=(0, 1), keepdims=True) * (1.0 / n)
    ratio_ref[...] = ratio
    mpm_ref[...] = jnp.sum(jax.nn.sigmoid(logits), axis=(0, 1), keepdims=True) * (1.0 / n)
    kre_ref[...] = 1.0 - 0.5 * ratio


def kernel(metric, W1, b1, W2, b2):
    tau_gate = 0.1
    theta_min = 0.0
    theta_max = 2.0
    B, T, C = metric.shape
    if T % 2 == 1:
        metric = metric[:, :-1, :]
        T = T - 1
    P = T // 2
    H = W1.shape[1]

    PT = 512
    while P % PT != 0:
        PT //= 2
    NT = P // PT
    TT = 2 * PT

    sim3, colsum = pl.pallas_call(
        functools.partial(_stage1_body, C=C),
        grid=(B, NT),
        in_specs=[pl.BlockSpec((1, TT, C), lambda b, t: (b, t, 0))],
        out_specs=[
            pl.BlockSpec((1, TT, 1), lambda b, t: (b, t, 0)),
            pl.BlockSpec((1, 1, C), lambda b, t: (b, 0, 0)),
        ],
        out_shape=[
            jax.ShapeDtypeStruct((B, T, 1), jnp.float32),
            jax.ShapeDtypeStruct((B, 1, C), jnp.float32),
        ],
    )(metric)

    sim = sim3.reshape(B, T)[:, ::2]
    colsum = colsum.reshape(B, C)

    outs = pl.pallas_call(
        functools.partial(_stage2_body, T=T, tau=tau_gate,
                          theta_min=theta_min, theta_max=theta_max),
        out_shape=[
            jax.ShapeDtypeStruct((B, P), jnp.float32),
            jax.ShapeDtypeStruct((B, P), jnp.float32),
            jax.ShapeDtypeStruct((B, 1), jnp.float32),
            jax.ShapeDtypeStruct((1, 1), jnp.float32),
            jax.ShapeDtypeStruct((1, 1), jnp.float32),
            jax.ShapeDtypeStruct((1, 1), jnp.float32),
        ],
    )(colsum, sim, W1, b1.reshape(1, H), W2, b2.reshape(1, 1))

    logits, maskf, theta2, ratio, mpm, kre = outs
    return (logits,
            maskf.astype(bool),
            theta2.reshape(B),
            ratio.reshape(()),
            mpm.reshape(()),
            kre.reshape(()))


# speedup vs baseline: 11.0843x; 1.1183x over previous
import functools

import jax
import jax.numpy as jnp
from jax.experimental import pallas as pl
from jax.experimental.pallas import tpu as pltpu


def _stage1_body(m_ref, sim_ref, cs_ref, *, C):
    t = pl.program_id(1)
    x = m_ref[0]
    tt = x.shape[0]
    xs = pltpu.roll(x, tt - 1, 0)
    dotf = jnp.sum(x * xs, axis=1, keepdims=True)
    n2 = jnp.sum(x * x, axis=1, keepdims=True)
    nrm = jnp.maximum(jnp.sqrt(n2), 1e-12)
    denom = nrm * pltpu.roll(nrm, tt - 1, 0)
    sim_ref[0] = dotf / denom
    g = jnp.sum(x, axis=0, keepdims=True)

    @pl.when(t == 0)
    def _init():
        cs_ref[0] = g

    @pl.when(t != 0)
    def _acc():
        cs_ref[0] += g


def _stage2_body(cs_ref, sim_ref, w1_ref, b1_ref, w2_ref, b2_ref,
                 logits_ref, mask_ref, theta_ref, ratio_ref, mpm_ref, kre_ref,
                 *, T, tau, theta_min, theta_max):
    g = cs_ref[...] * (1.0 / T)
    h = jnp.dot(g, w1_ref[...], preferred_element_type=jnp.float32) + b1_ref[...]
    h = 0.5 * h * (1.0 + jax.lax.erf(h * jnp.float32(0.7071067811865476)))
    t2 = jnp.dot(h, w2_ref[...], preferred_element_type=jnp.float32) + b2_ref[...]
    theta = theta_min + (theta_max - theta_min) * jax.nn.sigmoid(t2)
    theta_ref[...] = theta
    logits = (sim_ref[...] - theta) / max(tau, 1e-6)
    logits_ref[...] = logits
    maskf = (logits >= 0).astype(jnp.float32)
    mask_ref[...] = maskf
    n = logits.shape[0] * logits.shape[1]
    ratio = jnp.sum(maskf, axis=(0, 1), keepdims=True) * (1.0 / n)
    ratio_ref[...] = ratio
    mpm_ref[...] = jnp.sum(jax.nn.sigmoid(logits), axis=(0, 1), keepdims=True) * (1.0 / n)
    kre_ref[...] = 1.0 - 0.5 * ratio


def kernel(metric, W1, b1, W2, b2):
    tau_gate = 0.1
    theta_min = 0.0
    theta_max = 2.0
    B, T, C = metric.shape
    if T % 2 == 1:
        metric = metric[:, :-1, :]
        T = T - 1
    P = T // 2
    H = W1.shape[1]

    PT = 1024
    while P % PT != 0:
        PT //= 2
    NT = P // PT
    TT = 2 * PT

    sim3, colsum = pl.pallas_call(
        functools.partial(_stage1_body, C=C),
        grid=(B, NT),
        in_specs=[pl.BlockSpec((1, TT, C), lambda b, t: (b, t, 0))],
        out_specs=[
            pl.BlockSpec((1, TT, 1), lambda b, t: (b, t, 0)),
            pl.BlockSpec((1, 1, C), lambda b, t: (b, 0, 0)),
        ],
        out_shape=[
            jax.ShapeDtypeStruct((B, T, 1), jnp.float32),
            jax.ShapeDtypeStruct((B, 1, C), jnp.float32),
        ],
    )(metric)

    sim = sim3.reshape(B, T)[:, ::2]
    colsum = colsum.reshape(B, C)

    outs = pl.pallas_call(
        functools.partial(_stage2_body, T=T, tau=tau_gate,
                          theta_min=theta_min, theta_max=theta_max),
        out_shape=[
            jax.ShapeDtypeStruct((B, P), jnp.float32),
            jax.ShapeDtypeStruct((B, P), jnp.float32),
            jax.ShapeDtypeStruct((B, 1), jnp.float32),
            jax.ShapeDtypeStruct((1, 1), jnp.float32),
            jax.ShapeDtypeStruct((1, 1), jnp.float32),
            jax.ShapeDtypeStruct((1, 1), jnp.float32),
        ],
    )(colsum, sim, W1, b1.reshape(1, H), W2, b2.reshape(1, 1))

    logits, maskf, theta2, ratio, mpm, kre = outs
    return (logits,
            maskf.astype(bool),
            theta2.reshape(B),
            ratio.reshape(()),
            mpm.reshape(()),
            kre.reshape(()))


# speedup vs baseline: 11.6884x; 1.0545x over previous
import functools

import jax
import jax.numpy as jnp
from jax.experimental import pallas as pl
from jax.experimental.pallas import tpu as pltpu


def _stage1_body(m_ref, sim_ref, cs_ref, *, C):
    t = pl.program_id(1)
    x = m_ref[0]
    tt = x.shape[0]
    xs = pltpu.roll(x, tt - 1, 0)
    dotf = jnp.sum(x * xs, axis=1, keepdims=True)
    n2 = jnp.sum(x * x, axis=1, keepdims=True)
    nrm = jnp.maximum(jnp.sqrt(n2), 1e-12)
    denom = nrm * pltpu.roll(nrm, tt - 1, 0)
    sim_ref[0] = dotf / denom
    g = jnp.sum(x, axis=0, keepdims=True)

    @pl.when(t == 0)
    def _init():
        cs_ref[0] = g

    @pl.when(t != 0)
    def _acc():
        cs_ref[0] += g


def _stage2_body(cs_ref, sim_ref, w1_ref, b1_ref, w2_ref, b2_ref,
                 logits_ref, mask_ref, theta_ref, ratio_ref, mpm_ref, kre_ref,
                 *, T, tau, theta_min, theta_max):
    g = cs_ref[...] * (1.0 / T)
    h = jnp.dot(g, w1_ref[...], preferred_element_type=jnp.float32) + b1_ref[...]
    h = 0.5 * h * (1.0 + jax.lax.erf(h * jnp.float32(0.7071067811865476)))
    t2 = jnp.dot(h, w2_ref[...], preferred_element_type=jnp.float32) + b2_ref[...]
    theta = theta_min + (theta_max - theta_min) * jax.nn.sigmoid(t2)
    theta_ref[...] = theta
    logits = (sim_ref[...] - theta) / max(tau, 1e-6)
    logits_ref[...] = logits
    maskf = (logits >= 0).astype(jnp.float32)
    mask_ref[...] = maskf
    n = logits.shape[0] * logits.shape[1]
    ratio = jnp.sum(maskf, axis=(0, 1), keepdims=True) * (1.0 / n)
    ratio_ref[...] = ratio
    mpm_ref[...] = jnp.sum(jax.nn.sigmoid(logits), axis=(0, 1), keepdims=True) * (1.0 / n)
    kre_ref[...] = 1.0 - 0.5 * ratio


def kernel(metric, W1, b1, W2, b2):
    tau_gate = 0.1
    theta_min = 0.0
    theta_max = 2.0
    B, T, C = metric.shape
    if T % 2 == 1:
        metric = metric[:, :-1, :]
        T = T - 1
    P = T // 2
    H = W1.shape[1]

    PT = 2048
    while P % PT != 0:
        PT //= 2
    NT = P // PT
    TT = 2 * PT

    sim3, colsum = pl.pallas_call(
        functools.partial(_stage1_body, C=C),
        grid=(B, NT),
        in_specs=[pl.BlockSpec((1, TT, C), lambda b, t: (b, t, 0))],
        out_specs=[
            pl.BlockSpec((1, TT, 1), lambda b, t: (b, t, 0)),
            pl.BlockSpec((1, 1, C), lambda b, t: (b, 0, 0)),
        ],
        out_shape=[
            jax.ShapeDtypeStruct((B, T, 1), jnp.float32),
            jax.ShapeDtypeStruct((B, 1, C), jnp.float32),
        ],
    )(metric)

    sim = sim3.reshape(B, T)[:, ::2]
    colsum = colsum.reshape(B, C)

    outs = pl.pallas_call(
        functools.partial(_stage2_body, T=T, tau=tau_gate,
                          theta_min=theta_min, theta_max=theta_max),
        out_shape=[
            jax.ShapeDtypeStruct((B, P), jnp.float32),
            jax.ShapeDtypeStruct((B, P), jnp.float32),
            jax.ShapeDtypeStruct((B, 1), jnp.float32),
            jax.ShapeDtypeStruct((1, 1), jnp.float32),
            jax.ShapeDtypeStruct((1, 1), jnp.float32),
            jax.ShapeDtypeStruct((1, 1), jnp.float32),
        ],
    )(colsum, sim, W1, b1.reshape(1, H), W2, b2.reshape(1, 1))

    logits, maskf, theta2, ratio, mpm, kre = outs
    return (logits,
            maskf.astype(bool),
            theta2.reshape(B),
            ratio.reshape(()),
            mpm.reshape(()),
            kre.reshape(()))
